# baseline (speedup 1.0000x reference)
import functools

import jax
import jax.numpy as jnp
from jax import lax
from jax.experimental import pallas as pl
from jax.experimental.pallas import tpu as pltpu

RHO_ = 5.0
BS_ = 256
NHASH_ = 64
NLAB_ = 14
NVALS_ = NLAB_ + 1
QB_ = 8


def _loss_body(h_ref, l_ref, out_ref, dgt_ref, e_ref, s_ref, num_ref):
    lab = l_ref[:]
    dgt = lax.dot_general(lab, lab, (((1,), (1,)), ((), ())),
                          preferred_element_type=jnp.float32)
    rows = lax.broadcasted_iota(jnp.int32, (BS_, BS_), 0)
    cols = lax.broadcasted_iota(jnp.int32, (BS_, BS_), 1)
    dgt = jnp.where(rows == cols, 0.0, dgt)
    h = h_ref[:]
    gram = lax.dot_general(h, h, (((1,), (1,)), ((), ())),
                           preferred_element_type=jnp.float32)
    dsim = 0.5 * (jnp.float32(NHASH_) - gram)
    dgt_ref[:] = dgt
    e_ref[:] = jnp.exp2(dgt)
    s_ref[:] = dsim

    cum_le = jnp.zeros((BS_, 1), jnp.float32)
    z = jnp.zeros((BS_, 1), jnp.float32)
    cnt = jnp.zeros((BS_, 1), jnp.float32)
    prev1 = jnp.zeros((BS_, 1), jnp.float32)
    prev2 = jnp.zeros((BS_, 1), jnp.float32)
    for v in range(NVALS_):
        cv = jnp.sum(jnp.where(dgt == jnp.float32(v), 1.0, 0.0),
                     axis=1, keepdims=True)
        c_lt = cum_le
        cum_le = cum_le + cv
        c_gt = jnp.float32(BS_) - cum_le
        z = z + (2.0 ** v) * cv * (c_lt - c_gt)
        cnt = cnt + cv * (prev1 + prev2)
        prev2 = prev1
        prev1 = cv

    def blk(i, carry):
        q = i * QB_
        d = dgt_ref[pl.ds(q, QB_), :]
        ee = e_ref[pl.ds(q, QB_), :]
        s = s_ref[pl.ds(q, QB_), :]
        diff = d[:, :, None] - d[:, None, :]
        keep = (diff >= 0.5) & (diff <= 2.5)
        rij2 = ee[:, :, None] - ee[:, None, :]
        trip = jnp.maximum(s[:, :, None] - s[:, None, :] + RHO_, 0.0)
        contrib = jnp.where(keep, rij2 * trip, 0.0)
        blknum = jnp.sum(contrib, axis=2)
        num_ref[pl.ds(q, QB_), :] = jnp.sum(blknum, axis=1, keepdims=True)
        return carry

    lax.fori_loop(0, BS_ // QB_, blk, 0)

    num = num_ref[:]
    per_idx = jnp.where(z > 0.0, num / jnp.where(z > 0.0, z, 1.0), 0.0)
    losses = jnp.sum(per_idx)
    counts = jnp.sum(cnt)
    total = jnp.where(counts > 0.0, losses / counts, losses)
    out_ref[:, :] = jnp.broadcast_to(total, (1, 1))


@jax.jit
def kernel(hash_features, labels):
    out = pl.pallas_call(
        _loss_body,
        out_shape=jax.ShapeDtypeStruct((1, 1), jnp.float32),
        scratch_shapes=[
            pltpu.VMEM((BS_, BS_), jnp.float32),
            pltpu.VMEM((BS_, BS_), jnp.float32),
            pltpu.VMEM((BS_, BS_), jnp.float32),
            pltpu.VMEM((BS_, 1), jnp.float32),
        ],
    )(hash_features, labels)
    return jnp.reshape(out, (1,))


# speedup vs baseline: 1.0754x; 1.0754x over previous
import functools

import jax
import jax.numpy as jnp
import numpy as np
from jax import lax
from jax.experimental import pallas as pl
from jax.experimental.pallas import tpu as pltpu

RHO_ = 5.0
BS_ = 256
NHASH_ = 64
NLAB_ = 14
NVALS_ = NLAB_ + 1
QB_ = 8

def _loss_body(h_ref, l_ref, out_ref, dgt_ref, sb_ref, spb_ref, num_ref):
    lab = l_ref[:]
    dgt = lax.dot_general(lab, lab, (((1,), (1,)), ((), ())),
                          preferred_element_type=jnp.float32)
    rows = lax.broadcasted_iota(jnp.int32, (BS_, BS_), 0)
    cols = lax.broadcasted_iota(jnp.int32, (BS_, BS_), 1)
    dgt = jnp.where(rows == cols, 0.0, dgt)
    h = h_ref[:]
    gram = lax.dot_general(h, h, (((1,), (1,)), ((), ())),
                           preferred_element_type=jnp.float32)
    dsim = 0.5 * (jnp.float32(NHASH_) - gram)
    dgt_ref[:] = dgt
    sb_ref[:] = dsim.astype(jnp.bfloat16)
    spb_ref[:] = (dsim + RHO_).astype(jnp.bfloat16)

    cum_le = jnp.zeros((BS_, 1), jnp.float32)
    z = jnp.zeros((BS_, 1), jnp.float32)
    cnt = jnp.zeros((BS_, 1), jnp.float32)
    prev1 = jnp.zeros((BS_, 1), jnp.float32)
    prev2 = jnp.zeros((BS_, 1), jnp.float32)
    for v in range(NVALS_):
        cv = jnp.sum(jnp.where(dgt == jnp.float32(v), 1.0, 0.0),
                     axis=1, keepdims=True)
        c_lt = cum_le
        cum_le = cum_le + cv
        c_gt = jnp.float32(BS_) - cum_le
        z = z + (2.0 ** v) * cv * (c_lt - c_gt)
        cnt = cnt + cv * (prev1 + prev2)
        prev2 = prev1
        prev1 = cv

    a_i = lax.broadcasted_iota(jnp.int32, (16, 16), 0).astype(jnp.float32)
    b_i = lax.broadcasted_iota(jnp.int32, (16, 16), 1).astype(jnp.float32)
    d_ab = a_i - b_i
    wmat = jnp.where((d_ab >= 0.5) & (d_ab <= 2.5),
                     jnp.exp2(a_i) - jnp.exp2(b_i), 0.0)
    vals16 = lax.broadcasted_iota(jnp.int32, (16, 1), 0).astype(jnp.float32)

    def per_block(blk, carry):
        q0 = blk * QB_
        d_blk = dgt_ref[pl.ds(q0, QB_), :]
        sp_blk = spb_ref[pl.ds(q0, QB_), :]
        s_blk = sb_ref[pl.ds(q0, QB_), :]
        r3 = jnp.maximum(sp_blk[:, :, None] - s_blk[:, None, :],
                         jnp.bfloat16(0.0))
        nums = []
        for qq in range(QB_):
            d_row = d_blk[qq:qq + 1, :]
            ut_f = jnp.where(d_row == vals16, 1.0, 0.0)
            ut_b = ut_f.astype(jnp.bfloat16)
            p = lax.dot_general(ut_b, r3[qq], (((1,), (0,)), ((), ())),
                                preferred_element_type=jnp.float32)
            t = lax.dot_general(p, ut_f, (((1,), (1,)), ((), ())),
                                preferred_element_type=jnp.float32)
            nums.append(jnp.broadcast_to(jnp.sum(wmat * t), (1, 1)))
        num_ref[pl.ds(q0, QB_), :] = jnp.concatenate(nums, axis=0)
        return carry

    lax.fori_loop(0, BS_ // QB_, per_block, 0)

    num = num_ref[:]
    per_idx = jnp.where(z > 0.0, num / jnp.where(z > 0.0, z, 1.0), 0.0)
    losses = jnp.sum(per_idx)
    counts = jnp.sum(cnt)
    total = jnp.where(counts > 0.0, losses / counts, losses)
    out_ref[:, :] = jnp.broadcast_to(total, (1, 1))


@jax.jit
def kernel(hash_features, labels):
    out = pl.pallas_call(
        _loss_body,
        out_shape=jax.ShapeDtypeStruct((1, 1), jnp.float32),
        scratch_shapes=[
            pltpu.VMEM((BS_, BS_), jnp.float32),
            pltpu.VMEM((BS_, BS_), jnp.bfloat16),
            pltpu.VMEM((BS_, BS_), jnp.bfloat16),
            pltpu.VMEM((BS_, 1), jnp.float32),
        ],
    )(hash_features, labels)
    return jnp.reshape(out, (1,))


# speedup vs baseline: 3.7359x; 3.4739x over previous
import functools

import jax
import jax.numpy as jnp
import numpy as np
from jax import lax
from jax.experimental import pallas as pl
from jax.experimental.pallas import tpu as pltpu

RHO_ = 5.0
BS_ = 256
NHASH_ = 64
NLAB_ = 14
NVALS_ = NLAB_ + 1
QB_ = 16

def _loss_body(h_ref, l_ref, out_ref, dgt_ref, sb_ref, spb_ref, num_ref):
    lab = l_ref[:]
    dgt = lax.dot_general(lab, lab, (((1,), (1,)), ((), ())),
                          preferred_element_type=jnp.float32)
    rows = lax.broadcasted_iota(jnp.int32, (BS_, BS_), 0)
    cols = lax.broadcasted_iota(jnp.int32, (BS_, BS_), 1)
    dgt = jnp.where(rows == cols, 0.0, dgt)
    h = h_ref[:]
    gram = lax.dot_general(h, h, (((1,), (1,)), ((), ())),
                           preferred_element_type=jnp.float32)
    dsim = 0.5 * (jnp.float32(NHASH_) - gram)
    dgt_ref[:] = dgt
    sb_ref[:] = dsim.astype(jnp.bfloat16)
    spb_ref[:] = (dsim + RHO_).astype(jnp.bfloat16)

    cum_le = jnp.zeros((BS_, 1), jnp.float32)
    z = jnp.zeros((BS_, 1), jnp.float32)
    cnt = jnp.zeros((BS_, 1), jnp.float32)
    prev1 = jnp.zeros((BS_, 1), jnp.float32)
    prev2 = jnp.zeros((BS_, 1), jnp.float32)
    for v in range(NVALS_):
        cv = jnp.sum(jnp.where(dgt == jnp.float32(v), 1.0, 0.0),
                     axis=1, keepdims=True)
        c_lt = cum_le
        cum_le = cum_le + cv
        c_gt = jnp.float32(BS_) - cum_le
        z = z + (2.0 ** v) * cv * (c_lt - c_gt)
        cnt = cnt + cv * (prev1 + prev2)
        prev2 = prev1
        prev1 = cv

    vals16 = lax.broadcasted_iota(jnp.int32, (16, 1), 0).astype(jnp.float32)
    vals16e = jnp.exp2(vals16)

    def per_block(blk, carry):
        q0 = blk * QB_
        d_blk = dgt_ref[pl.ds(q0, QB_), :]
        e_blk = jnp.exp2(d_blk)
        sp_blk = spb_ref[pl.ds(q0, QB_), :]
        s_blk = sb_ref[pl.ds(q0, QB_), :]
        r3 = jnp.maximum(sp_blk[:, :, None] - s_blk[:, None, :],
                         jnp.bfloat16(0.0))
        nums = []
        for qq in range(QB_):
            d_row = d_blk[qq:qq + 1, :]
            ut_b = (d_row == vals16).astype(jnp.bfloat16)
            p = lax.dot_general(ut_b, r3[qq], (((1,), (0,)), ((), ())),
                                preferred_element_type=jnp.float32)
            adiff = vals16 - d_row
            v = jnp.where((adiff >= 0.5) & (adiff <= 2.5),
                          vals16e - e_blk[qq:qq + 1, :], 0.0)
            nums.append(jnp.broadcast_to(jnp.sum(p * v), (1, 1)))
        num_ref[pl.ds(q0, QB_), :] = jnp.concatenate(nums, axis=0)
        return carry

    lax.fori_loop(0, BS_ // QB_, per_block, 0)

    num = num_ref[:]
    per_idx = jnp.where(z > 0.0, num / jnp.where(z > 0.0, z, 1.0), 0.0)
    losses = jnp.sum(per_idx)
    counts = jnp.sum(cnt)
    total = jnp.where(counts > 0.0, losses / counts, losses)
    out_ref[:, :] = jnp.broadcast_to(total, (1, 1))


@jax.jit
def kernel(hash_features, labels):
    out = pl.pallas_call(
        _loss_body,
        out_shape=jax.ShapeDtypeStruct((1, 1), jnp.float32),
        scratch_shapes=[
            pltpu.VMEM((BS_, BS_), jnp.float32),
            pltpu.VMEM((BS_, BS_), jnp.bfloat16),
            pltpu.VMEM((BS_, BS_), jnp.bfloat16),
            pltpu.VMEM((BS_, 1), jnp.float32),
        ],
    )(hash_features, labels)
    return jnp.reshape(out, (1,))


# speedup vs baseline: 4.5139x; 1.2083x over previous
import functools

import jax
import jax.numpy as jnp
import numpy as np
from jax import lax
from jax.experimental import pallas as pl
from jax.experimental.pallas import tpu as pltpu

RHO_ = 5.0
BS_ = 256
NHASH_ = 64
NLAB_ = 14
NVALS_ = NLAB_ + 1
QB_ = 32

def _loss_body(h_ref, l_ref, out_ref, dgt_ref, sb_ref, spb_ref, num_ref):
    lab = l_ref[:]
    dgt = lax.dot_general(lab, lab, (((1,), (1,)), ((), ())),
                          preferred_element_type=jnp.float32)
    rows = lax.broadcasted_iota(jnp.int32, (BS_, BS_), 0)
    cols = lax.broadcasted_iota(jnp.int32, (BS_, BS_), 1)
    dgt = jnp.where(rows == cols, 0.0, dgt)
    h = h_ref[:]
    gram = lax.dot_general(h, h, (((1,), (1,)), ((), ())),
                           preferred_element_type=jnp.float32)
    dsim = 0.5 * (jnp.float32(NHASH_) - gram)
    dgt_ref[:] = dgt
    sb_ref[:] = dsim.astype(jnp.bfloat16)
    spb_ref[:] = (dsim + RHO_).astype(jnp.bfloat16)

    cum_le = jnp.zeros((BS_, 1), jnp.float32)
    z = jnp.zeros((BS_, 1), jnp.float32)
    cnt = jnp.zeros((BS_, 1), jnp.float32)
    prev1 = jnp.zeros((BS_, 1), jnp.float32)
    prev2 = jnp.zeros((BS_, 1), jnp.float32)
    for v in range(NVALS_):
        cv = jnp.sum(jnp.where(dgt == jnp.float32(v), 1.0, 0.0),
                     axis=1, keepdims=True)
        c_lt = cum_le
        cum_le = cum_le + cv
        c_gt = jnp.float32(BS_) - cum_le
        z = z + (2.0 ** v) * cv * (c_lt - c_gt)
        cnt = cnt + cv * (prev1 + prev2)
        prev2 = prev1
        prev1 = cv

    vals16 = lax.broadcasted_iota(jnp.int32, (16, 1), 0).astype(jnp.float32)
    vals16e = jnp.exp2(vals16)

    def per_block(blk, carry):
        q0 = blk * QB_
        d_blk = dgt_ref[pl.ds(q0, QB_), :]
        e_blk = jnp.exp2(d_blk)
        sp_blk = spb_ref[pl.ds(q0, QB_), :]
        s_blk = sb_ref[pl.ds(q0, QB_), :]
        r3 = jnp.maximum(sp_blk[:, :, None] - s_blk[:, None, :],
                         jnp.bfloat16(0.0))
        nums = []
        for qq in range(QB_):
            d_row = d_blk[qq:qq + 1, :]
            ut_b = (d_row == vals16).astype(jnp.bfloat16)
            p = lax.dot_general(ut_b, r3[qq], (((1,), (0,)), ((), ())),
                                preferred_element_type=jnp.float32)
            adiff = vals16 - d_row
            v = jnp.where((adiff >= 0.5) & (adiff <= 2.5),
                          vals16e - e_blk[qq:qq + 1, :], 0.0)
            nums.append(jnp.broadcast_to(jnp.sum(p * v), (1, 1)))
        num_ref[pl.ds(q0, QB_), :] = jnp.concatenate(nums, axis=0)
        return carry

    lax.fori_loop(0, BS_ // QB_, per_block, 0)

    num = num_ref[:]
    per_idx = jnp.where(z > 0.0, num / jnp.where(z > 0.0, z, 1.0), 0.0)
    losses = jnp.sum(per_idx)
    counts = jnp.sum(cnt)
    total = jnp.where(counts > 0.0, losses / counts, losses)
    out_ref[:, :] = jnp.broadcast_to(total, (1, 1))


@jax.jit
def kernel(hash_features, labels):
    out = pl.pallas_call(
        _loss_body,
        out_shape=jax.ShapeDtypeStruct((1, 1), jnp.float32),
        scratch_shapes=[
            pltpu.VMEM((BS_, BS_), jnp.float32),
            pltpu.VMEM((BS_, BS_), jnp.bfloat16),
            pltpu.VMEM((BS_, BS_), jnp.bfloat16),
            pltpu.VMEM((BS_, 1), jnp.float32),
        ],
    )(hash_features, labels)
    return jnp.reshape(out, (1,))


# speedup vs baseline: 4.9985x; 1.1073x over previous
import functools

import jax
import jax.numpy as jnp
import numpy as np
from jax import lax
from jax.experimental import pallas as pl
from jax.experimental.pallas import tpu as pltpu

RHO_ = 5.0
BS_ = 256
NHASH_ = 64
NLAB_ = 14
NVALS_ = NLAB_ + 1
QB_ = 64

def _loss_body(h_ref, l_ref, out_ref, dgt_ref, sb_ref, spb_ref, num_ref):
    lab = l_ref[:]
    dgt = lax.dot_general(lab, lab, (((1,), (1,)), ((), ())),
                          preferred_element_type=jnp.float32)
    rows = lax.broadcasted_iota(jnp.int32, (BS_, BS_), 0)
    cols = lax.broadcasted_iota(jnp.int32, (BS_, BS_), 1)
    dgt = jnp.where(rows == cols, 0.0, dgt)
    h = h_ref[:]
    gram = lax.dot_general(h, h, (((1,), (1,)), ((), ())),
                           preferred_element_type=jnp.float32)
    dsim = 0.5 * (jnp.float32(NHASH_) - gram)
    dgt_ref[:] = dgt
    sb_ref[:] = dsim.astype(jnp.bfloat16)
    spb_ref[:] = (dsim + RHO_).astype(jnp.bfloat16)

    cum_le = jnp.zeros((BS_, 1), jnp.float32)
    z = jnp.zeros((BS_, 1), jnp.float32)
    cnt = jnp.zeros((BS_, 1), jnp.float32)
    prev1 = jnp.zeros((BS_, 1), jnp.float32)
    prev2 = jnp.zeros((BS_, 1), jnp.float32)
    for v in range(NVALS_):
        cv = jnp.sum(jnp.where(dgt == jnp.float32(v), 1.0, 0.0),
                     axis=1, keepdims=True)
        c_lt = cum_le
        cum_le = cum_le + cv
        c_gt = jnp.float32(BS_) - cum_le
        z = z + (2.0 ** v) * cv * (c_lt - c_gt)
        cnt = cnt + cv * (prev1 + prev2)
        prev2 = prev1
        prev1 = cv

    vals16 = lax.broadcasted_iota(jnp.int32, (16, 1), 0).astype(jnp.float32)
    vals16e = jnp.exp2(vals16)

    def per_block(blk, carry):
        q0 = blk * QB_
        d_blk = dgt_ref[pl.ds(q0, QB_), :]
        e_blk = jnp.exp2(d_blk)
        sp_blk = spb_ref[pl.ds(q0, QB_), :]
        s_blk = sb_ref[pl.ds(q0, QB_), :]
        r3 = jnp.maximum(sp_blk[:, :, None] - s_blk[:, None, :],
                         jnp.bfloat16(0.0))
        nums = []
        for qq in range(QB_):
            d_row = d_blk[qq:qq + 1, :]
            ut_b = (d_row == vals16).astype(jnp.bfloat16)
            p = lax.dot_general(ut_b, r3[qq], (((1,), (0,)), ((), ())),
                                preferred_element_type=jnp.float32)
            adiff = vals16 - d_row
            v = jnp.where((adiff >= 0.5) & (adiff <= 2.5),
                          vals16e - e_blk[qq:qq + 1, :], 0.0)
            nums.append(jnp.broadcast_to(jnp.sum(p * v), (1, 1)))
        num_ref[pl.ds(q0, QB_), :] = jnp.concatenate(nums, axis=0)
        return carry

    lax.fori_loop(0, BS_ // QB_, per_block, 0)

    num = num_ref[:]
    per_idx = jnp.where(z > 0.0, num / jnp.where(z > 0.0, z, 1.0), 0.0)
    losses = jnp.sum(per_idx)
    counts = jnp.sum(cnt)
    total = jnp.where(counts > 0.0, losses / counts, losses)
    out_ref[:, :] = jnp.broadcast_to(total, (1, 1))


@jax.jit
def kernel(hash_features, labels):
    out = pl.pallas_call(
        _loss_body,
        out_shape=jax.ShapeDtypeStruct((1, 1), jnp.float32),
        scratch_shapes=[
            pltpu.VMEM((BS_, BS_), jnp.float32),
            pltpu.VMEM((BS_, BS_), jnp.bfloat16),
            pltpu.VMEM((BS_, BS_), jnp.bfloat16),
            pltpu.VMEM((BS_, 1), jnp.float32),
        ],
    )(hash_features, labels)
    return jnp.reshape(out, (1,))


# speedup vs baseline: 5.4485x; 1.0900x over previous
import functools

import jax
import jax.numpy as jnp
import numpy as np
from jax import lax
from jax.experimental import pallas as pl
from jax.experimental.pallas import tpu as pltpu

RHO_ = 5.0
BS_ = 256
NHASH_ = 64
NLAB_ = 14
NVALS_ = NLAB_ + 1
QB_ = 128

def _loss_body(h_ref, l_ref, out_ref, dgt_ref, sb_ref, spb_ref, num_ref):
    lab = l_ref[:]
    dgt = lax.dot_general(lab, lab, (((1,), (1,)), ((), ())),
                          preferred_element_type=jnp.float32)
    rows = lax.broadcasted_iota(jnp.int32, (BS_, BS_), 0)
    cols = lax.broadcasted_iota(jnp.int32, (BS_, BS_), 1)
    dgt = jnp.where(rows == cols, 0.0, dgt)
    h = h_ref[:]
    gram = lax.dot_general(h, h, (((1,), (1,)), ((), ())),
                           preferred_element_type=jnp.float32)
    dsim = 0.5 * (jnp.float32(NHASH_) - gram)
    dgt_ref[:] = dgt
    sb_ref[:] = dsim.astype(jnp.bfloat16)
    spb_ref[:] = (dsim + RHO_).astype(jnp.bfloat16)

    cum_le = jnp.zeros((BS_, 1), jnp.float32)
    z = jnp.zeros((BS_, 1), jnp.float32)
    cnt = jnp.zeros((BS_, 1), jnp.float32)
    prev1 = jnp.zeros((BS_, 1), jnp.float32)
    prev2 = jnp.zeros((BS_, 1), jnp.float32)
    for v in range(NVALS_):
        cv = jnp.sum(jnp.where(dgt == jnp.float32(v), 1.0, 0.0),
                     axis=1, keepdims=True)
        c_lt = cum_le
        cum_le = cum_le + cv
        c_gt = jnp.float32(BS_) - cum_le
        z = z + (2.0 ** v) * cv * (c_lt - c_gt)
        cnt = cnt + cv * (prev1 + prev2)
        prev2 = prev1
        prev1 = cv

    vals16 = lax.broadcasted_iota(jnp.int32, (16, 1), 0).astype(jnp.float32)
    vals16e = jnp.exp2(vals16)

    def per_block(blk, carry):
        q0 = blk * QB_
        d_blk = dgt_ref[pl.ds(q0, QB_), :]
        e_blk = jnp.exp2(d_blk)
        sp_blk = spb_ref[pl.ds(q0, QB_), :]
        s_blk = sb_ref[pl.ds(q0, QB_), :]
        r3 = jnp.maximum(sp_blk[:, :, None] - s_blk[:, None, :],
                         jnp.bfloat16(0.0))
        nums = []
        for qq in range(QB_):
            d_row = d_blk[qq:qq + 1, :]
            ut_b = (d_row == vals16).astype(jnp.bfloat16)
            p = lax.dot_general(ut_b, r3[qq], (((1,), (0,)), ((), ())),
                                preferred_element_type=jnp.float32)
            adiff = vals16 - d_row
            v = jnp.where((adiff >= 0.5) & (adiff <= 2.5),
                          vals16e - e_blk[qq:qq + 1, :], 0.0)
            nums.append(jnp.broadcast_to(jnp.sum(p * v), (1, 1)))
        num_ref[pl.ds(q0, QB_), :] = jnp.concatenate(nums, axis=0)
        return carry

    lax.fori_loop(0, BS_ // QB_, per_block, 0)

    num = num_ref[:]
    per_idx = jnp.where(z > 0.0, num / jnp.where(z > 0.0, z, 1.0), 0.0)
    losses = jnp.sum(per_idx)
    counts = jnp.sum(cnt)
    total = jnp.where(counts > 0.0, losses / counts, losses)
    out_ref[:, :] = jnp.broadcast_to(total, (1, 1))


@jax.jit
def kernel(hash_features, labels):
    out = pl.pallas_call(
        _loss_body,
        out_shape=jax.ShapeDtypeStruct((1, 1), jnp.float32),
        scratch_shapes=[
            pltpu.VMEM((BS_, BS_), jnp.float32),
            pltpu.VMEM((BS_, BS_), jnp.bfloat16),
            pltpu.VMEM((BS_, BS_), jnp.bfloat16),
            pltpu.VMEM((BS_, 1), jnp.float32),
        ],
    )(hash_features, labels)
    return jnp.reshape(out, (1,))
